# baseline (speedup 1.0000x reference)
import functools

import jax
import jax.numpy as jnp
from jax import lax
from jax.experimental import pallas as pl
from jax.experimental.pallas import tpu as pltpu
from jax.experimental.pallas import tpu_sc as plsc

_CH = 128
_NW = 32


def _make_sc_probe(U):
    nchunks = U // _CH
    full = nchunks // _NW
    outer = (full - 1) // 2
    mesh = plsc.VectorSubcoreMesh(core_axis_name="c", subcore_axis_name="s")

    @functools.partial(
        pl.kernel,
        out_type=jax.ShapeDtypeStruct((U, 128), jnp.float32),
        mesh=mesh,
        scratch_types=[
            pltpu.VMEM((2, _CH, 128), jnp.float32),
            pltpu.SemaphoreType.DMA,
            pltpu.SemaphoreType.DMA,
        ],
    )
    def sc_k(ids_hbm, out_hbm, rows, sem_s0, sem_s1):
        wid = lax.axis_index("s") * 2 + lax.axis_index("c")
        sems = [sem_s0, sem_s1]

        def body(jj, carry):
            for b in range(2):
                c = (jj * 2 + b) * _NW + wid

                @pl.when(jj > 0)
                def _():
                    pltpu.make_async_copy(
                        rows.at[b], out_hbm.at[pl.ds(0, _CH)], sems[b]
                    ).wait()

                pltpu.async_copy(
                    rows.at[b], out_hbm.at[pl.ds(c * _CH, _CH)], sems[b]
                )
            return carry

        lax.fori_loop(0, outer, body, 0)
        for b in range(2):
            pltpu.make_async_copy(
                rows.at[b], out_hbm.at[pl.ds(0, _CH)], sems[b]
            ).wait()

        c = (outer * 2) * _NW + wid
        pltpu.async_copy(rows.at[0], out_hbm.at[pl.ds(c * _CH, _CH)], sem_s0).wait()

    return sc_k


def kernel(input_ids, embedding, species_embedding):
    B, S, T = input_ids.shape
    H = embedding.shape[1]
    TP = 56
    U = B * S * (TP // 8) * 2 * 8
    ids_flat = input_ids.reshape(B * S * T)
    sc_k = _make_sc_probe(U)
    buf = sc_k(ids_flat)
    out = (
        buf.reshape(B, S, TP // 8, 2, 8, 128)
        .transpose(0, 1, 2, 4, 3, 5)
        .reshape(B, S, TP, H)[:, :, :T, :]
    )
    return out


# speedup vs baseline: 1.2588x; 1.2588x over previous
import functools

import jax
import jax.numpy as jnp
from jax import lax
from jax.experimental import pallas as pl
from jax.experimental.pallas import tpu as pltpu
from jax.experimental.pallas import tpu_sc as plsc

_L = 16
_CH = 128
_NW = 32
_NBUF = 2
_B_SC = 16


def _comb_body(emb_ref, sp_ref, out_ref):
    out_ref[...] = sp_ref[...][:, None, :] + emb_ref[...][None, :, :]


def _make_comb(embedding, species_embedding):
    S, H = species_embedding.shape
    comb = pl.pallas_call(
        _comb_body,
        out_shape=jax.ShapeDtypeStruct((S, 3, H), jnp.float32),
    )(embedding, species_embedding)
    return comb.reshape(S * 3, H)


def _tc_body(ids_ref, emb_ref, sp_ref, out_ref):
    ids = ids_ref[...][..., None]
    e0 = emb_ref[0]
    e1 = emb_ref[1]
    e2 = emb_ref[2]
    sp = sp_ref[...][None, :, None, :]
    out_ref[...] = jnp.where(ids == 0, e0, jnp.where(ids == 1, e1, e2)) + sp


def _tc_part(input_ids, embedding, species_embedding):
    Bt, S, T = input_ids.shape
    H = embedding.shape[1]
    return pl.pallas_call(
        _tc_body,
        grid=(Bt,),
        in_specs=[
            pl.BlockSpec((1, S, T), lambda b: (b, 0, 0)),
            pl.BlockSpec((3, H), lambda b: (0, 0)),
            pl.BlockSpec((S, H), lambda b: (0, 0)),
        ],
        out_specs=pl.BlockSpec((1, S, T, H), lambda b: (b, 0, 0, 0)),
        out_shape=jax.ShapeDtypeStruct((Bt, S, T, H), jnp.float32),
    )(input_ids, embedding, species_embedding)


def _make_sc_kernel(N, H, T, S):
    nchunks = N // _CH
    full = nchunks // _NW
    tail = nchunks - full * _NW
    outer = full // _NBUF
    extra = full - outer * _NBUF
    mesh = plsc.VectorSubcoreMesh(core_axis_name="c", subcore_axis_name="s")

    @functools.partial(
        pl.kernel,
        out_type=jax.ShapeDtypeStruct((N, H), jnp.float32),
        mesh=mesh,
        scratch_types=[
            pltpu.VMEM((_CH,), jnp.int32),
            pltpu.VMEM((_CH,), jnp.int32),
            pltpu.VMEM((_NBUF, _CH, H), jnp.float32),
            pltpu.SemaphoreType.DMA,
            pltpu.SemaphoreType.DMA,
            pltpu.SemaphoreType.DMA,
        ],
    )
    def sc_k(ids_hbm, comb_hbm, out_hbm, idsv, keys, rows, sem_g, sem_s0, sem_s1):
        wid = lax.axis_index("s") * 2 + lax.axis_index("c")
        iota = lax.iota(jnp.int32, _L)
        sems = [sem_s0, sem_s1]

        def load_keys(c):
            base = c * _CH
            pltpu.sync_copy(ids_hbm.at[pl.ds(base, _CH)], idsv)
            for i in range(_CH // _L):
                n = base + i * _L + iota
                s = lax.rem(lax.div(n, jnp.int32(T)), jnp.int32(S))
                keys[pl.ds(i * _L, _L)] = idsv[pl.ds(i * _L, _L)] + s * 3

        def gather_scatter(c, b, sem_b, first):
            load_keys(c)

            @pl.when(jnp.logical_not(first))
            def _():
                pltpu.make_async_copy(
                    rows.at[b], out_hbm.at[pl.ds(0, _CH)], sem_b
                ).wait()

            pltpu.async_copy(comb_hbm.at[keys], rows.at[b], sem_g).wait()
            pltpu.async_copy(rows.at[b], out_hbm.at[pl.ds(c * _CH, _CH)], sem_b)

        def body(jj, carry):
            for b in range(_NBUF):
                j = jj * _NBUF + b
                gather_scatter(j * _NW + wid, b, sems[b], jj == 0)
            return carry

        lax.fori_loop(0, outer, body, 0)
        if extra:
            gather_scatter((outer * _NBUF) * _NW + wid, 0, sems[0],
                           jnp.bool_(outer == 0))

        @pl.when(wid < tail)
        def _():
            gather_scatter(full * _NW + wid, 1, sems[1], jnp.bool_(full < 2))

        for b in range(_NBUF):
            pltpu.make_async_copy(
                rows.at[b], out_hbm.at[pl.ds(0, _CH)], sems[b]
            ).wait()

    return sc_k


def kernel(input_ids, embedding, species_embedding):
    B, S, T = input_ids.shape
    H = embedding.shape[1]
    comb = _make_comb(embedding, species_embedding)

    b_tc = B - _B_SC
    out_tc = _tc_part(input_ids[:b_tc], embedding, species_embedding)

    n_sc = _B_SC * S * T
    ids_sc = input_ids[b_tc:].reshape(n_sc)
    sc_k = _make_sc_kernel(n_sc, H, T, S)
    out_sc = sc_k(ids_sc, comb).reshape(_B_SC, S, T, H)

    return jnp.concatenate([out_tc, out_sc], axis=0)
